# baseline (speedup 1.0000x reference)
import jax
import jax.numpy as jnp
from jax import lax
from jax.experimental import pallas as pl
from jax.experimental.pallas import tpu as pltpu
from jax.experimental.pallas import tpu_sc as plsc

NL = 100000
CD = 64
PW = 128
NLP = NL // 2
NB = 16384
LRC = 0.05
THETA = 0.3

CH = 8192
CHB = 13
CHP = CH // 2
NTRASH = 128
NCH = 13
NFULL = NL // CH
LASTP = NLP - NFULL * CHP
EV_T = NB // 16
ROWS_T = CHP // 16
ACC_ROWS = CHP + NTRASH
BLK = 64
NBLK = EV_T // BLK

_mesh = plsc.VectorSubcoreMesh(core_axis_name="c", subcore_axis_name="s")
_params = pltpu.CompilerParams(use_tc_tiling_on_sc=True,
                               needs_layout_passes=False)

_i32 = jnp.int32
_f32 = jnp.float32


def _fused_body(w0_hbm, wlp_hbm, idx_hbm, con_hbm, phon_hbm, out_hbm,
                idxS, idxF, evid, conB, abuf, phonB,
                sidxB, gidxB, vidxB, smem,
                acc, dacc, sem, sem2, sem3):
    c = lax.axis_index("c")
    s = lax.axis_index("s")
    ev0 = s * EV_T
    lane = lax.iota(_i32, 16)

    pltpu.sync_copy(idx_hbm.at[pl.ds(ev0, EV_T)], idxS.at[pl.ds(0, EV_T)])

    def cntf(g, cnts):
        ck = jnp.right_shift(idxS[pl.ds(g * 16, 16)], CHB)
        return tuple(cnts[k] + jnp.sum(jnp.where(ck == k, 1, 0))
                     for k in range(NCH))
    cnts = lax.fori_loop(0, EV_T // 16, cntf, (_i32(0),) * NCH)

    running = _i32(0)
    for k in range(NCH):
        smem[k] = running
        smem[32 + k] = running
        running = running + cnts[k]

    lane0 = lane == 0

    def fill(e, _):
        iv = idxS[pl.ds(e, 16)][0]
        k = jnp.right_shift(iv, CHB)
        p = smem[32 + k]
        smem[32 + k] = p + 1
        r = jnp.full((16,), jnp.right_shift(p, 6), _i32)
        cc = jnp.full((16,), p & (BLK - 1), _i32)
        plsc.store_scatter(idxF, [r, cc], jnp.full((16,), iv, _i32),
                           mask=lane0)
        plsc.store_scatter(evid, [r, cc], jnp.full((16,), ev0 + e, _i32),
                           mask=lane0)
        return 0
    lax.fori_loop(0, EV_T, fill, 0)

    def build_sidx(b, k):
        def bld(g, _):
            iv = idxF[b, pl.ds(g * 16, 16)]
            inb = jnp.right_shift(iv, CHB) == k
            li = jnp.where(inb, jnp.right_shift(iv & (CH - 1), 1),
                           CHP + ((iv + lane) & (NTRASH - 1)))
            sidxB[0, pl.ds(g * 16, 16)] = li
            return 0
        lax.fori_loop(0, BLK // 16, bld, 0)

    def do_chunk(k, rows_t, tail=0):
        kbase = k * CHP
        lo = smem[k]
        hi = smem[32 + k]
        b0 = jnp.right_shift(lo, 6)
        b1 = jnp.right_shift(hi + BLK - 1, 6)

        pltpu.sync_copy(w0_hbm.at[pl.ds(kbase + s * rows_t, rows_t)],
                        acc.at[pl.ds(s * rows_t, rows_t)])
        pltpu.sync_copy(wlp_hbm.at[pl.ds(kbase + s * rows_t, rows_t)],
                        dacc.at[pl.ds(s * rows_t, rows_t)])
        if tail:
            @pl.when(s == 0)
            def _():
                pltpu.sync_copy(
                    w0_hbm.at[pl.ds(kbase + 16 * rows_t, tail)],
                    acc.at[pl.ds(16 * rows_t, tail)])
                pltpu.sync_copy(
                    wlp_hbm.at[pl.ds(kbase + 16 * rows_t, tail)],
                    dacc.at[pl.ds(16 * rows_t, tail)])
        plsc.subcore_barrier()

        def sblk(b, _):
            cpc = pltpu.async_copy(con_hbm.at[evid.at[b]], conB, sem3)
            build_sidx(b, k)
            cpc.wait()
            pltpu.sync_copy(conB, acc.at[sidxB.at[0]], add=True)
            return 0
        lax.fori_loop(b0, b1, sblk, 0)
        plsc.subcore_barrier()

        def gblk(b, _):
            def bld2(g, _):
                iv = idxF[b, pl.ds(g * 16, 16)]
                loc = jnp.right_shift(iv & (CH - 1), 1)
                inb = jnp.right_shift(iv, CHB) == k
                gidxB[0, pl.ds(g * 16, 16)] = loc
                vidxB[0, pl.ds(g * 16, 16)] = jnp.where(
                    inb, loc, CHP + ((iv + lane) & (NTRASH - 1)))
                return 0
            lax.fori_loop(0, BLK // 16, bld2, 0)
            cpa = pltpu.async_copy(acc.at[gidxB.at[0]], abuf, sem)
            cpc = pltpu.async_copy(con_hbm.at[evid.at[b]], conB, sem2)
            cpp = pltpu.async_copy(phon_hbm.at[evid.at[b]], phonB, sem3)
            cpa.wait()
            cpc.wait()
            cpp.wait()

            def dotf(e, _):
                sl = pl.ds(0, 16)
                r = abuf[e, sl] * conB[e, sl]
                for j in range(1, PW // 16):
                    sl = pl.ds(j * 16, 16)
                    r = r + abuf[e, sl] * conB[e, sl]
                act = jnp.sum(r)
                scale = jnp.where(act > THETA * LRC, _f32(LRC), _f32(0.0))
                for j in range(PW // 16):
                    sl = pl.ds(j * 16, 16)
                    phonB[e, sl] = phonB[e, sl] * scale
                return 0
            lax.fori_loop(0, BLK, dotf, 0)
            pltpu.sync_copy(phonB, dacc.at[vidxB.at[0]], add=True)
            return 0
        lax.fori_loop(b0, b1, gblk, 0)
        plsc.subcore_barrier()

        pltpu.sync_copy(dacc.at[pl.ds(s * rows_t, rows_t)],
                        out_hbm.at[pl.ds(kbase + s * rows_t, rows_t)])
        if tail:
            @pl.when(s == 0)
            def _():
                pltpu.sync_copy(
                    dacc.at[pl.ds(16 * rows_t, tail)],
                    out_hbm.at[pl.ds(kbase + 16 * rows_t, tail)])

    def full_chunk(kk, _):
        do_chunk(2 * kk + c, ROWS_T)
        return 0
    lax.fori_loop(0, NFULL // 2, full_chunk, 0)

    @pl.when(c == 0)
    def _():
        do_chunk(_i32(NFULL), (LASTP // 16) & ~7,
                 LASTP - 16 * ((LASTP // 16) & ~7))


_sc_fused = pl.kernel(
    _fused_body,
    out_type=jax.ShapeDtypeStruct((NLP, PW), _f32),
    mesh=_mesh,
    compiler_params=_params,
    scratch_types=[
        pltpu.VMEM((EV_T + 16,), _i32),
        pltpu.VMEM((NBLK, BLK), _i32),
        pltpu.VMEM((NBLK, BLK), _i32),
        pltpu.VMEM((BLK, PW), _f32),
        pltpu.VMEM((BLK, PW), _f32),
        pltpu.VMEM((BLK, PW), _f32),
        pltpu.VMEM((1, BLK), _i32),
        pltpu.VMEM((1, BLK), _i32),
        pltpu.VMEM((1, BLK), _i32),
        pltpu.SMEM((64,), _i32),
        pltpu.VMEM_SHARED((ACC_ROWS, PW), _f32),
        pltpu.VMEM_SHARED((ACC_ROWS, PW), _f32),
        pltpu.SemaphoreType.DMA,
        pltpu.SemaphoreType.DMA,
        pltpu.SemaphoreType.DMA,
    ],
)


def kernel(W_C_to_L, W_L_to_P, idx, concept, phon):
    idx = idx.astype(_i32)
    w0p = W_C_to_L.reshape(NLP, PW)
    wlpp = W_L_to_P.reshape(NLP, PW)
    hi = (idx & 1)[:, None] == 1
    lc = concept * _f32(LRC)
    z = jnp.zeros_like(concept)
    c4 = jnp.where(hi, jnp.concatenate([z, lc], axis=1),
                   jnp.concatenate([lc, z], axis=1))
    p4 = jnp.where(hi, jnp.concatenate([z, phon], axis=1),
                   jnp.concatenate([phon, z], axis=1))
    outp = _sc_fused(w0p, wlpp, idx, c4, p4)
    return outp.reshape(NL, CD)
